# baseline (speedup 1.0000x reference)
import functools

import jax
import jax.numpy as jnp
from jax import lax
from jax.experimental import pallas as pl
from jax.experimental.pallas import tpu as pltpu
from jax.experimental.pallas import tpu_sc as plsc

_NUM_CORES = 2
_NUM_SUBCORES = 16
_NW = _NUM_CORES * _NUM_SUBCORES

_NB = 16384
_S = 50
_D = 64
_V = 1000000
_B = _NB * _S
_B_PER_W = _B // _NW
_CHUNK = 512
_N_CHUNKS = _B_PER_W // _CHUNK

_mesh = plsc.VectorSubcoreMesh(
    core_axis_name="c",
    subcore_axis_name="s",
    num_cores=_NUM_CORES,
    num_subcores=_NUM_SUBCORES,
)


@functools.partial(
    pl.kernel,
    out_type=jax.ShapeDtypeStruct((_B, _D), jnp.float32),
    mesh=_mesh,
    compiler_params=pltpu.CompilerParams(use_tc_tiling_on_sc=False),
    scratch_types=[
        pltpu.VMEM((_B_PER_W,), jnp.int32),
        pltpu.VMEM((_CHUNK, _D), jnp.float32),
        pltpu.VMEM((_CHUNK, _D), jnp.float32),
        pltpu.SemaphoreType.DMA,
        pltpu.SemaphoreType.DMA,
        pltpu.SemaphoreType.DMA,
        pltpu.SemaphoreType.DMA,
    ],
)
def _gather_kernel(table_hbm, idx_hbm, out_hbm, idx_v, rows0, rows1,
                   gsem0, gsem1, wsem0, wsem1):
    wid = lax.axis_index("s") * _NUM_CORES + lax.axis_index("c")
    base = wid * _B_PER_W
    pltpu.sync_copy(idx_hbm.at[pl.ds(base, _B_PER_W)], idx_v)

    bufs = (rows0, rows1)
    gsems = (gsem0, gsem1)
    wsems = (wsem0, wsem1)

    def start_gather(i, slot):
        pltpu.async_copy(
            table_hbm.at[idx_v.at[pl.ds(i * _CHUNK, _CHUNK)]],
            bufs[slot], gsems[slot])

    def wait_gather(slot):
        pltpu.make_async_copy(
            table_hbm.at[idx_v.at[pl.ds(0, _CHUNK)]],
            bufs[slot], gsems[slot]).wait()

    def start_wb(i, slot):
        pltpu.async_copy(
            bufs[slot], out_hbm.at[pl.ds(base + i * _CHUNK, _CHUNK)],
            wsems[slot])

    def wait_wb(slot):
        pltpu.make_async_copy(
            bufs[slot], out_hbm.at[pl.ds(base, _CHUNK)], wsems[slot]).wait()

    start_gather(0, 0)
    start_gather(1, 1)
    wait_gather(0)
    start_wb(0, 0)

    def pair_body(p, carry):
        j = 1 + 2 * p
        wait_wb(0)
        start_gather(j + 1, 0)
        wait_gather(1)
        start_wb(j, 1)
        wait_wb(1)
        start_gather(j + 2, 1)
        wait_gather(0)
        start_wb(j + 1, 0)
        return carry

    lax.fori_loop(0, (_N_CHUNKS - 2) // 2, pair_body, 0)

    wait_gather(1)
    start_wb(_N_CHUNKS - 1, 1)
    wait_wb(0)
    wait_wb(1)


_WBLK = 2048


def _w_tc_body(in_ref, out_ref):
    t = in_ref[...].T.reshape(_WBLK // 2, 2, _D)
    out_ref[...] = jnp.concatenate([t[:, 0, :], t[:, 1, :]], axis=1)


_w_tc = pl.pallas_call(
    _w_tc_body,
    out_shape=jax.ShapeDtypeStruct((_V // 2, 2 * _D), jnp.float32),
    grid=(_V // _WBLK + 1,),
    in_specs=[pl.BlockSpec((_D, _WBLK), lambda k: (0, k))],
    out_specs=pl.BlockSpec((_WBLK // 2, 2 * _D), lambda k: (k, 0)),
)

_SP = _S // 2
_TBLOCKS = _SP * (_NB // 128)
_TBLK_PER_W = _TBLOCKS // _NW


@functools.partial(
    pl.kernel,
    out_type=jax.ShapeDtypeStruct((_S * _D, _NB), jnp.float32),
    mesh=_mesh,
    compiler_params=pltpu.CompilerParams(
        use_tc_tiling_on_sc=True, needs_layout_passes=False),
    scratch_types=[
        pltpu.VMEM((128,), jnp.int32),
        pltpu.VMEM((128, 128), jnp.float32),
        pltpu.VMEM((128, 128), jnp.float32),
        pltpu.VMEM((128, 128), jnp.float32),
        pltpu.VMEM((128, 128), jnp.float32),
        pltpu.SemaphoreType.DMA,
        pltpu.SemaphoreType.DMA,
        pltpu.SemaphoreType.DMA,
        pltpu.SemaphoreType.DMA,
    ],
)
def _o_sc_kernel(gpair_hbm, out_hbm, idx_v, in0, in1, ob0, ob1,
                 gsem0, gsem1, wsem0, wsem1):
    wid = lax.axis_index("s") * _NUM_CORES + lax.axis_index("c")
    ins = (in0, in1)
    obs = (ob0, ob1)
    gsems = (gsem0, gsem1)
    wsems = (wsem0, wsem1)
    iota = lax.iota(jnp.int32, 16)
    io25 = iota * 25
    rows_g = [iota + 16 * g for g in range(8)]

    def blk(k):
        t = wid * _TBLK_PER_W + k
        return t // (_NB // 128), t % (_NB // 128)

    def start_gather(k, slot):
        sp, bb = blk(k)
        def build(g, carry):
            idx_v[pl.ds(g * 16, 16)] = io25 + ((bb * 128 + g * 16) * 25 + sp)
            return carry
        lax.fori_loop(0, 8, build, 0)
        pltpu.async_copy(gpair_hbm.at[idx_v], ins[slot], gsems[slot])

    def wait_gather(slot):
        pltpu.make_async_copy(gpair_hbm.at[idx_v], ins[slot],
                              gsems[slot]).wait()

    def transpose(slot):
        src, ob = ins[slot], obs[slot]

        def brow(b, carry):
            bs = jnp.full((16,), b, jnp.int32)
            for g in range(8):
                v = src[b, pl.ds(g * 16, 16)]
                plsc.store_scatter(ob, [rows_g[g], bs], v)
            return carry

        lax.fori_loop(0, 128, brow, 0)

    def start_wb(k, slot):
        sp, bb = blk(k)
        pltpu.async_copy(
            obs[slot],
            out_hbm.at[pl.ds(sp * 128, 128), pl.ds(bb * 128, 128)],
            wsems[slot])

    def wait_wb(slot):
        pltpu.make_async_copy(
            obs[slot], out_hbm.at[pl.ds(0, 128), pl.ds(0, 128)],
            wsems[slot]).wait()

    start_gather(0, 0)
    wait_gather(0)
    start_gather(1, 1)
    transpose(0)
    start_wb(0, 0)

    def pair_body(p, carry):
        j = 1 + 2 * p
        wait_gather(1)
        transpose(1)
        wait_wb(0)
        start_gather(j + 1, 0)
        start_wb(j, 1)
        wait_gather(0)
        transpose(0)
        wait_wb(1)
        start_gather(j + 2, 1)
        start_wb(j + 1, 0)
        return carry

    lax.fori_loop(0, (_TBLK_PER_W - 2) // 2, pair_body, 0)

    wait_gather(1)
    transpose(1)
    wait_wb(0)
    start_wb(_TBLK_PER_W - 1, 1)
    wait_wb(1)


def kernel(token_ids, weight):
    flat = token_ids.reshape(-1)
    wlin = _w_tc(weight.T).reshape(_V, _D)
    out_flat = _gather_kernel(wlin, flat)
    o2d = _o_sc_kernel(out_flat.reshape(_B // 2, 2 * _D))
    return jnp.transpose(o2d.reshape(_S, _D, _NB), (2, 0, 1))


# speedup vs baseline: 1.9677x; 1.9677x over previous
import functools

import jax
import jax.numpy as jnp
from jax import lax
from jax.experimental import pallas as pl
from jax.experimental.pallas import tpu as pltpu
from jax.experimental.pallas import tpu_sc as plsc

_NUM_CORES = 2
_NUM_SUBCORES = 16
_NW = _NUM_CORES * _NUM_SUBCORES

_NB = 16384
_S = 50
_D = 64
_V = 1000000
_B = _NB * _S
_B_PER_W = _B // _NW
_CHUNK = 512
_N_CHUNKS = _B_PER_W // _CHUNK

_mesh = plsc.VectorSubcoreMesh(
    core_axis_name="c",
    subcore_axis_name="s",
    num_cores=_NUM_CORES,
    num_subcores=_NUM_SUBCORES,
)


@functools.partial(
    pl.kernel,
    out_type=jax.ShapeDtypeStruct((_B, _D), jnp.float32),
    mesh=_mesh,
    compiler_params=pltpu.CompilerParams(use_tc_tiling_on_sc=False),
    scratch_types=[
        pltpu.VMEM((_B_PER_W,), jnp.int32),
        pltpu.VMEM((_CHUNK, _D), jnp.float32),
        pltpu.VMEM((_CHUNK, _D), jnp.float32),
        pltpu.SemaphoreType.DMA,
        pltpu.SemaphoreType.DMA,
        pltpu.SemaphoreType.DMA,
        pltpu.SemaphoreType.DMA,
    ],
)
def _gather_kernel(table_hbm, idx_hbm, out_hbm, idx_v, rows0, rows1,
                   gsem0, gsem1, wsem0, wsem1):
    wid = lax.axis_index("s") * _NUM_CORES + lax.axis_index("c")
    base = wid * _B_PER_W
    pltpu.sync_copy(idx_hbm.at[pl.ds(base, _B_PER_W)], idx_v)

    bufs = (rows0, rows1)
    gsems = (gsem0, gsem1)
    wsems = (wsem0, wsem1)

    def start_gather(i, slot):
        pltpu.async_copy(
            table_hbm.at[idx_v.at[pl.ds(i * _CHUNK, _CHUNK)]],
            bufs[slot], gsems[slot])

    def wait_gather(slot):
        pltpu.make_async_copy(
            table_hbm.at[idx_v.at[pl.ds(0, _CHUNK)]],
            bufs[slot], gsems[slot]).wait()

    def start_wb(i, slot):
        pltpu.async_copy(
            bufs[slot], out_hbm.at[pl.ds(base + i * _CHUNK, _CHUNK)],
            wsems[slot])

    def wait_wb(slot):
        pltpu.make_async_copy(
            bufs[slot], out_hbm.at[pl.ds(base, _CHUNK)], wsems[slot]).wait()

    start_gather(0, 0)
    start_gather(1, 1)
    wait_gather(0)
    start_wb(0, 0)

    def pair_body(p, carry):
        j = 1 + 2 * p
        wait_wb(0)
        start_gather(j + 1, 0)
        wait_gather(1)
        start_wb(j, 1)
        wait_wb(1)
        start_gather(j + 2, 1)
        wait_gather(0)
        start_wb(j + 1, 0)
        return carry

    lax.fori_loop(0, (_N_CHUNKS - 2) // 2, pair_body, 0)

    wait_gather(1)
    start_wb(_N_CHUNKS - 1, 1)
    wait_wb(0)
    wait_wb(1)


_WBLK = 2048


def _w_tc_body(in_ref, out_ref):
    t = in_ref[...].T.reshape(_WBLK // 2, 2, _D)
    out_ref[...] = jnp.concatenate([t[:, 0, :], t[:, 1, :]], axis=1)


_w_tc = pl.pallas_call(
    _w_tc_body,
    out_shape=jax.ShapeDtypeStruct((_V // 2, 2 * _D), jnp.float32),
    grid=(_V // _WBLK + 1,),
    in_specs=[pl.BlockSpec((_D, _WBLK), lambda k: (0, k))],
    out_specs=pl.BlockSpec((_WBLK // 2, 2 * _D), lambda k: (k, 0)),
)

_SP = _S // 2


def _o_tc_body(in_ref, out_ref):
    t = in_ref[...].reshape(128, _SP, 128)
    for sp in range(_SP):
        out_ref[pl.ds(sp * 128, 128), :] = t[:, sp, :].T


_o_tc = pl.pallas_call(
    _o_tc_body,
    out_shape=jax.ShapeDtypeStruct((_S * _D, _NB), jnp.float32),
    grid=(_NB // 128,),
    in_specs=[pl.BlockSpec((128 * _SP, 128), lambda m: (m, 0))],
    out_specs=pl.BlockSpec((_S * _D, 128), lambda m: (0, m)),
)


def kernel(token_ids, weight):
    flat = token_ids.reshape(-1)
    wlin = _w_tc(weight.T).reshape(_V, _D)
    out_flat = _gather_kernel(wlin, flat)
    o2d = _o_tc(out_flat.reshape(_B // 2, 2 * _D))
    return jnp.transpose(o2d.reshape(_S, _D, _NB), (2, 0, 1))
